# baseline (speedup 1.0000x reference)
import functools

import jax
import jax.numpy as jnp
from jax import lax
from jax.experimental import pallas as pl
from jax.experimental.pallas import tpu as pltpu
from jax.experimental.pallas import tpu_sc as plsc

Q = 1024
D = 128
N = 100000
NPAD = 100352
DB = 2048
NBLK = NPAD // DB
CHUNK = 128
NCHUNK = NPAD // CHUNK
NCHUNK_PAD = 896
K = 10
NCLASS = 1000
NW = 32
QPW = Q // NW
MAXL = 800
INF = float("inf")


def _dist_body(x_ref, data_ref, out_ref, min_ref, *, qq):
    x = x_ref[...]
    db = data_ref[...]
    x_sq = jnp.sum(x * x, axis=1, keepdims=True)
    d_sq = jnp.sum(db * db, axis=1)[None, :]
    p = lax.dot_general(
        x, db, (((1,), (1,)), ((), ())),
        preferred_element_type=jnp.float32,
        precision=lax.Precision.DEFAULT,
    )
    d2 = x_sq - 2.0 * p + d_sq
    j = pl.program_id(0)
    cols = j * DB + lax.broadcasted_iota(jnp.int32, (qq, DB), 1)
    d2 = jnp.where(cols < N, d2, INF)
    d3 = d2.reshape(qq, DB // CHUNK, CHUNK)
    out_ref[...] = d3
    mins16 = jnp.min(d3, axis=2)
    mblk = jnp.concatenate(
        [mins16, jnp.full((qq, CHUNK - DB // CHUNK), INF, jnp.float32)],
        axis=1)
    min_ref[...] = mblk[None, :, :]


def _distances(x, data_pad, qq):
    return pl.pallas_call(
        functools.partial(_dist_body, qq=qq),
        grid=(NBLK,),
        in_specs=[
            pl.BlockSpec((qq, D), lambda j: (0, 0)),
            pl.BlockSpec((DB, D), lambda j: (j, 0)),
        ],
        out_specs=[
            pl.BlockSpec((qq, DB // CHUNK, CHUNK), lambda j: (0, j, 0)),
            pl.BlockSpec((1, qq, CHUNK), lambda j: (j, 0, 0)),
        ],
        out_shape=[
            jax.ShapeDtypeStruct((qq, NCHUNK, CHUNK), jnp.float32),
            jax.ShapeDtypeStruct((NBLK, qq, CHUNK), jnp.float32),
        ],
    )(x, data_pad)


def _sc_body(d2f_hbm, mins_hbm, tgt_hbm, out_hbm,
             minb, clist, gbuf, idxall, tgall, tpbuf, predbuf, dma_sem,
             *, qpw):
    cid = lax.axis_index("c")
    sid = lax.axis_index("s")
    wid = sid * 2 + cid
    base_q = wid * qpw
    lane = lax.iota(jnp.int32, 16)
    ones = jnp.ones((16,), jnp.int32)
    zeros = jnp.zeros((16,), jnp.int32)

    QH = min(16, qpw)

    def merge16(bv, bi, candv, candi):
        candv, candi = plsc.sort_key_val(candv, candi)
        rbv = lax.rev(bv, (0,))
        rbi = lax.rev(bi, (0,))
        take = candv < rbv
        lov = jnp.where(take, candv, rbv)
        loi = jnp.where(take, candi, rbi)
        sv, si = plsc.sort_key_val(lov, loi)
        return sv, si

    def per_query(qi2, h):
        qi = h * QH + qi2
        q = base_q + qi

        def a_body(c, carry):
            bv, bi, thr = carry
            mv = minb[c, qi2, pl.ds(0, 16)]
            hit = jnp.max(jnp.where(mv < thr, ones, zeros))

            def do(cc):
                abv, abi = cc
                candv = jnp.where(mv < thr, mv, INF)
                abv, abi = merge16(abv, abi, candv, c * 16 + lane)
                t_s = jnp.min(jnp.where(lane == K - 1, abv, INF))
                return abv, abi, jnp.full((16,), t_s, jnp.float32)

            return lax.cond(hit > 0, do, lambda cc: (cc[0], cc[1], thr),
                            (bv, bi))

        av, ai, athr = lax.fori_loop(
            0, NBLK, a_body,
            (jnp.full((16,), INF, jnp.float32), jnp.zeros((16,), jnp.int32),
             jnp.full((16,), INF, jnp.float32)))
        t_s = jnp.min(jnp.where(lane == K - 1, av, INF))
        T = jnp.full((16,), t_s, jnp.float32)

        def b_body(c, offset):
            mv = minb[c, qi2, pl.ds(0, 16)]
            mask = mv <= T
            mi = jnp.where(mask, ones, zeros)
            cnt = jnp.sum(mi)

            @pl.when(cnt > 0)
            def _():
                pos = offset + plsc.cumsum(mi) - 1
                plsc.store_scatter(clist, [pos],
                                   q * NCHUNK + c * 16 + lane, mask=mask)

            return offset + cnt

        cntq = lax.fori_loop(0, NBLK, b_body, jnp.int32(0))
        nrounds = (cntq + 15) // 16

        pad_idx = q * NCHUNK + (NCHUNK - 1)

        def c_body(r, carry):
            bv, bi, thr = carry
            idxv = clist[pl.ds(r * 16, 16)]
            valid = (r * 16 + lane) < cntq
            idxv = jnp.where(valid, idxv, pad_idx)
            pltpu.async_copy(d2f_hbm.at[idxv], gbuf, dma_sem).wait()
            chunkv = idxv - q * NCHUNK

            for i in range(16):
                ci = jnp.max(jnp.where(lane == i, chunkv, zeros))
                vs = [gbuf[i, pl.ds(v * 16, 16)] for v in range(8)]
                m = vs[0]
                for v in range(1, 8):
                    m = jnp.minimum(m, vs[v])
                rhit = jnp.max(jnp.where(m < thr, ones, zeros))

                def row(cc, ci=ci, vs=vs):
                    rbv, rbi = cc
                    for v in range(8):
                        vj = vs[v]
                        hit = jnp.max(jnp.where(vj < thr, ones, zeros))

                        def mrg(bc, vj=vj, v=v, ci=ci):
                            mbv, mbi = bc
                            candv = jnp.where(vj < thr, vj, INF)
                            candi = ci * CHUNK + v * 16 + lane
                            return merge16(mbv, mbi, candv, candi)

                        rbv, rbi = lax.cond(hit > 0, mrg, lambda bc: bc,
                                            (rbv, rbi))
                    t2 = jnp.min(jnp.where(lane == K - 1, rbv, INF))
                    return rbv, rbi, jnp.full((16,), t2, jnp.float32)

                bv, bi, thr = lax.cond(rhit > 0, row,
                                       lambda cc: (cc[0], cc[1], thr),
                                       (bv, bi))
            return bv, bi, thr

        bestv, besti, thr = lax.fori_loop(
            0, nrounds, c_body,
            (jnp.full((16,), INF, jnp.float32), jnp.zeros((16,), jnp.int32),
             jnp.full((16,), INF, jnp.float32)))

        idxall[pl.ds(qi * 16, 16)] = besti
        return h

    for h in range(qpw // QH):
        pltpu.sync_copy(
            mins_hbm.at[:, pl.ds(base_q + h * QH, QH), :], minb)
        lax.fori_loop(0, QH, per_query, h)

    copies = [
        pltpu.async_copy(tgt_hbm.at[idxall.at[pl.ds(b * 128, 128)]],
                         tgall.at[pl.ds(b * 128, 128)], dma_sem)
        for b in range(qpw * 16 // 128)
    ]
    for cp in copies:
        cp.wait()

    def vote(qi, _):
        t = tgall[pl.ds(qi * 16, 16)]
        tp = jnp.where(lane < K, t, NCLASS + lane)
        tpbuf[...] = tp
        count = jnp.ones((16,), jnp.int32)
        for r in range(1, K):
            perm = lax.rem(lane + r, jnp.full((16,), K, jnp.int32))
            tr = plsc.load_gather(tpbuf, [perm])
            count = count + jnp.where(tp == tr, ones, zeros)
        score = jnp.where(lane < K, count * 1024 + (1023 - tp), -ones)
        smax = jnp.max(score)
        pred = (1023 - lax.rem(smax, 1024)).astype(jnp.float32)
        plsc.store_scatter(predbuf, [jnp.full((16,), qi, jnp.int32)],
                           jnp.full((16,), pred, jnp.float32), mask=lane == 0)
        return 0

    lax.fori_loop(0, qpw, vote, 0)
    pltpu.sync_copy(predbuf, out_hbm.at[pl.ds(base_q, qpw)])


def _topk_mode(d2flat, mins, targets, qq):
    qpw = qq // NW
    mesh = plsc.VectorSubcoreMesh(core_axis_name="c", subcore_axis_name="s")
    f = functools.partial(
        pl.kernel,
        out_type=jax.ShapeDtypeStruct((qq,), jnp.float32),
        scratch_types=[
            pltpu.VMEM((NBLK, min(16, qpw), CHUNK), jnp.float32),
            pltpu.VMEM((MAXL,), jnp.int32),
            pltpu.VMEM((16, CHUNK), jnp.float32),
            pltpu.VMEM((qpw * 16,), jnp.int32),
            pltpu.VMEM((qpw * 16,), jnp.int32),
            pltpu.VMEM((16,), jnp.int32),
            pltpu.VMEM((qpw,), jnp.float32),
            pltpu.SemaphoreType.DMA,
        ],
        mesh=mesh,
        compiler_params=pltpu.CompilerParams(needs_layout_passes=False),
    )(functools.partial(_sc_body, qpw=qpw))
    return f(d2flat, mins, targets)


def kernel(x, data, targets):
    qh = Q // 4
    outs = []
    for i in range(4):
        xs = lax.slice(x, (i * qh, 0), ((i + 1) * qh, D))
        d2, mins3 = _distances(xs, data, qh)
        d2flat = d2.reshape(qh * NCHUNK, CHUNK)
        outs.append(_topk_mode(d2flat, mins3, targets, qh))
    return jnp.concatenate(outs)


# speedup vs baseline: 1.0701x; 1.0701x over previous
import functools

import jax
import jax.numpy as jnp
from jax import lax
from jax.experimental import pallas as pl
from jax.experimental.pallas import tpu as pltpu
from jax.experimental.pallas import tpu_sc as plsc

Q = 1024
D = 128
N = 100000
NPAD = 100352
DB = 2048
NBLK = NPAD // DB
CHUNK = 128
NCHUNK = NPAD // CHUNK
NCHUNK_PAD = 896
K = 10
NCLASS = 1000
NW = 32
QPW = Q // NW
MAXL = 800
INF = float("inf")


def _dist_body(x_ref, data_ref, out_ref, min_ref, *, qq):
    x = x_ref[...]
    db = data_ref[...]
    x_sq = jnp.sum(x * x, axis=1, keepdims=True)
    d_sq = jnp.sum(db * db, axis=1)[None, :]
    p = lax.dot_general(
        x, db, (((1,), (1,)), ((), ())),
        preferred_element_type=jnp.float32,
        precision=lax.Precision.DEFAULT,
    )
    d2 = x_sq - 2.0 * p + d_sq
    j = pl.program_id(0)
    cols = j * DB + lax.broadcasted_iota(jnp.int32, (qq, DB), 1)
    d2 = jnp.where(cols < N, d2, INF)
    d3 = d2.reshape(qq, DB // CHUNK, CHUNK)
    out_ref[...] = d3
    mins16 = jnp.min(d3, axis=2)
    mblk = jnp.concatenate(
        [mins16, jnp.full((qq, CHUNK - DB // CHUNK), INF, jnp.float32)],
        axis=1)
    min_ref[...] = mblk[None, :, :]


def _distances(x, data_pad, qq):
    return pl.pallas_call(
        functools.partial(_dist_body, qq=qq),
        grid=(NBLK,),
        in_specs=[
            pl.BlockSpec((qq, D), lambda j: (0, 0)),
            pl.BlockSpec((DB, D), lambda j: (j, 0)),
        ],
        out_specs=[
            pl.BlockSpec((qq, DB // CHUNK, CHUNK), lambda j: (0, j, 0)),
            pl.BlockSpec((1, qq, CHUNK), lambda j: (j, 0, 0)),
        ],
        out_shape=[
            jax.ShapeDtypeStruct((qq, NCHUNK, CHUNK), jnp.float32),
            jax.ShapeDtypeStruct((NBLK, qq, CHUNK), jnp.float32),
        ],
    )(x, data_pad)


def _sc_body(d2f_hbm, mins_hbm, tgt_hbm, out_hbm,
             minb, clist, gbuf, idxall, tgall, tpbuf, predbuf, dma_sem,
             *, qpw):
    cid = lax.axis_index("c")
    sid = lax.axis_index("s")
    wid = sid * 2 + cid
    base_q = wid * qpw
    lane = lax.iota(jnp.int32, 16)
    ones = jnp.ones((16,), jnp.int32)
    zeros = jnp.zeros((16,), jnp.int32)

    QH = min(16, qpw)

    def merge16(bv, bi, candv, candi):
        candv, candi = plsc.sort_key_val(candv, candi)
        rbv = lax.rev(bv, (0,))
        rbi = lax.rev(bi, (0,))
        take = candv < rbv
        lov = jnp.where(take, candv, rbv)
        loi = jnp.where(take, candi, rbi)
        sv, si = plsc.sort_key_val(lov, loi)
        return sv, si

    def per_query(qi2, h):
        qi = h * QH + qi2
        q = base_q + qi

        def a_body(c, carry):
            bv, bi, thr = carry
            mv = minb[c, qi2, pl.ds(0, 16)]
            hit = jnp.max(jnp.where(mv < thr, ones, zeros))

            def do(cc):
                abv, abi = cc
                candv = jnp.where(mv < thr, mv, INF)
                abv, abi = merge16(abv, abi, candv, c * 16 + lane)
                t_s = jnp.min(jnp.where(lane == K - 1, abv, INF))
                return abv, abi, jnp.full((16,), t_s, jnp.float32)

            return lax.cond(hit > 0, do, lambda cc: (cc[0], cc[1], thr),
                            (bv, bi))

        av, ai, athr = lax.fori_loop(
            0, NBLK, a_body,
            (jnp.full((16,), INF, jnp.float32), jnp.zeros((16,), jnp.int32),
             jnp.full((16,), INF, jnp.float32)))
        t_s = jnp.min(jnp.where(lane == K - 1, av, INF))
        T = jnp.full((16,), t_s, jnp.float32)

        def b_body(c, offset):
            mv = minb[c, qi2, pl.ds(0, 16)]
            mask = mv <= T
            mi = jnp.where(mask, ones, zeros)
            cnt = jnp.sum(mi)

            @pl.when(cnt > 0)
            def _():
                pos = offset + plsc.cumsum(mi) - 1
                plsc.store_scatter(clist, [pos],
                                   q * NCHUNK + c * 16 + lane, mask=mask)

            return offset + cnt

        cntq = lax.fori_loop(0, NBLK, b_body, jnp.int32(0))
        nrounds = (cntq + 15) // 16

        pad_idx = q * NCHUNK + (NCHUNK - 1)

        def c_body(r, carry):
            bv, bi, thr = carry
            idxv = clist[pl.ds(r * 16, 16)]
            valid = (r * 16 + lane) < cntq
            idxv = jnp.where(valid, idxv, pad_idx)
            pltpu.async_copy(d2f_hbm.at[idxv], gbuf, dma_sem).wait()
            chunkv = idxv - q * NCHUNK

            for i in range(16):
                ci = jnp.max(jnp.where(lane == i, chunkv, zeros))
                vs = [gbuf[i, pl.ds(v * 16, 16)] for v in range(8)]
                m = vs[0]
                for v in range(1, 8):
                    m = jnp.minimum(m, vs[v])
                rhit = jnp.max(jnp.where(m < thr, ones, zeros))

                def row(cc, ci=ci, vs=vs):
                    rbv, rbi = cc
                    for v in range(8):
                        vj = vs[v]
                        hit = jnp.max(jnp.where(vj < thr, ones, zeros))

                        def mrg(bc, vj=vj, v=v, ci=ci):
                            mbv, mbi = bc
                            candv = jnp.where(vj < thr, vj, INF)
                            candi = ci * CHUNK + v * 16 + lane
                            return merge16(mbv, mbi, candv, candi)

                        rbv, rbi = lax.cond(hit > 0, mrg, lambda bc: bc,
                                            (rbv, rbi))
                    t2 = jnp.min(jnp.where(lane == K - 1, rbv, INF))
                    return rbv, rbi, jnp.full((16,), t2, jnp.float32)

                bv, bi, thr = lax.cond(rhit > 0, row,
                                       lambda cc: (cc[0], cc[1], thr),
                                       (bv, bi))
            return bv, bi, thr

        bestv, besti, thr = lax.fori_loop(
            0, nrounds, c_body,
            (jnp.full((16,), INF, jnp.float32), jnp.zeros((16,), jnp.int32),
             jnp.full((16,), INF, jnp.float32)))

        idxall[pl.ds(qi * 16, 16)] = besti
        return h

    for h in range(qpw // QH):
        pltpu.sync_copy(
            mins_hbm.at[:, pl.ds(base_q + h * QH, QH), :], minb)
        lax.fori_loop(0, QH, per_query, h)

    copies = [
        pltpu.async_copy(tgt_hbm.at[idxall.at[pl.ds(b * 128, 128)]],
                         tgall.at[pl.ds(b * 128, 128)], dma_sem)
        for b in range(qpw * 16 // 128)
    ]
    for cp in copies:
        cp.wait()

    def vote(qi, _):
        t = tgall[pl.ds(qi * 16, 16)]
        tp = jnp.where(lane < K, t, NCLASS + lane)
        tpbuf[...] = tp
        count = jnp.ones((16,), jnp.int32)
        for r in range(1, K):
            perm = lax.rem(lane + r, jnp.full((16,), K, jnp.int32))
            tr = plsc.load_gather(tpbuf, [perm])
            count = count + jnp.where(tp == tr, ones, zeros)
        score = jnp.where(lane < K, count * 1024 + (1023 - tp), -ones)
        smax = jnp.max(score)
        pred = (1023 - lax.rem(smax, 1024)).astype(jnp.float32)
        plsc.store_scatter(predbuf, [jnp.full((16,), qi, jnp.int32)],
                           jnp.full((16,), pred, jnp.float32), mask=lane == 0)
        return 0

    lax.fori_loop(0, qpw, vote, 0)
    pltpu.sync_copy(predbuf, out_hbm.at[pl.ds(base_q, qpw)])


def _topk_mode(d2flat, mins, targets, qq):
    qpw = qq // NW
    mesh = plsc.VectorSubcoreMesh(core_axis_name="c", subcore_axis_name="s")
    f = functools.partial(
        pl.kernel,
        out_type=jax.ShapeDtypeStruct((qq,), jnp.float32),
        scratch_types=[
            pltpu.VMEM((NBLK, min(16, qpw), CHUNK), jnp.float32),
            pltpu.VMEM((MAXL,), jnp.int32),
            pltpu.VMEM((16, CHUNK), jnp.float32),
            pltpu.VMEM((qpw * 16,), jnp.int32),
            pltpu.VMEM((qpw * 16,), jnp.int32),
            pltpu.VMEM((16,), jnp.int32),
            pltpu.VMEM((qpw,), jnp.float32),
            pltpu.SemaphoreType.DMA,
        ],
        mesh=mesh,
        compiler_params=pltpu.CompilerParams(needs_layout_passes=False),
    )(functools.partial(_sc_body, qpw=qpw))
    return f(d2flat, mins, targets)


def kernel(x, data, targets):
    qh = Q // 2
    outs = []
    for i in range(2):
        xs = lax.slice(x, (i * qh, 0), ((i + 1) * qh, D))
        d2, mins3 = _distances(xs, data, qh)
        d2flat = d2.reshape(qh * NCHUNK, CHUNK)
        outs.append(_topk_mode(d2flat, mins3, targets, qh))
    return jnp.concatenate(outs)
